# baseline (speedup 1.0000x reference)
import jax
import jax.numpy as jnp
from jax import lax
from jax.experimental import pallas as pl
from jax.experimental.pallas import tpu as pltpu
from jax.experimental.pallas import tpu_sc as plsc

N = 10000
D = 128
E = 320000
NC = 2
NS = 16
NW = NC * NS
EPW = E // NW
C = 125
CP = 128
NCHUNK = EPW // C
NP = N + 8
ND = 10016
RPT = 624
TAIL = N - NS * RPT
SUB = 24
NSUB = RPT // SUB
CV = CP // 16


def _make_sc_agg(with_deg):
    out_type = [jax.ShapeDtypeStruct((NC * N, D), jnp.float32)]
    scratch = [
        pltpu.VMEM_SHARED((NP, D), jnp.float32),
        pltpu.VMEM((2, CP), jnp.int32),
        pltpu.VMEM((2, CP), jnp.int32),
        pltpu.VMEM((CP, D), jnp.float32),
        pltpu.VMEM((CP, D), jnp.float32),
        pltpu.VMEM((SUB, D), jnp.float32),
        pltpu.VMEM((SUB, D), jnp.float32),
        pltpu.SemaphoreType.DMA,
        pltpu.SemaphoreType.DMA,
        pltpu.SemaphoreType.DMA,
        pltpu.SemaphoreType.DMA,
        pltpu.SemaphoreType.DMA,
    ]
    if with_deg:
        out_type.append(jax.ShapeDtypeStruct((NW, N), jnp.float32))
        scratch.append(pltpu.VMEM((ND,), jnp.float32))

    mesh = plsc.VectorSubcoreMesh(core_axis_name="c", subcore_axis_name="s")

    def body(x_hbm, sd_hbm, zrow_hbm,
             agg_out, deg_out, agg_sh, sd0, sd1,
             rows0, rows1, zbuf, zbuf2, gsem0, gsem1, isem0, isem1,
             wsem, degloc=None):
        cid = lax.axis_index("c")
        sid = lax.axis_index("s")
        wid = sid * NC + cid

        pltpu.sync_copy(sd_hbm.at[wid].at[0], sd0)
        g0 = pltpu.async_copy(x_hbm.at[sd0.at[0]], rows0, gsem0)
        pltpu.sync_copy(sd_hbm.at[wid].at[1], sd1)
        g1 = pltpu.async_copy(x_hbm.at[sd1.at[0]], rows1, gsem1)

        pltpu.sync_copy(zrow_hbm, zbuf)

        @pl.loop(0, NSUB)
        def _(j):
            pltpu.async_copy(zbuf, agg_sh.at[pl.ds(sid * RPT + j * SUB, SUB)],
                             wsem)

        @pl.when(sid == 0)
        def _():
            pltpu.sync_copy(zbuf.at[pl.ds(0, TAIL)],
                            agg_sh.at[pl.ds(NS * RPT, TAIL)])

        if with_deg:
            zv = jnp.zeros((16,), jnp.float32)

            @pl.loop(0, ND // 16)
            def _(k):
                degloc[pl.ds(k * 16, 16)] = zv

        @pl.loop(0, NSUB)
        def _(j):
            pltpu.make_async_copy(
                zbuf, agg_sh.at[pl.ds(sid * RPT, SUB)], wsem).wait()

        plsc.subcore_barrier()

        ones16 = jnp.ones((16,), jnp.float32)

        def deg_update(sd):
            for g in range(CV):
                idx = sd[1, pl.ds(g * 16, 16)]
                plsc.addupdate_scatter(degloc, [idx], ones16)

        @pl.loop(0, NCHUNK, step=2)
        def _(i):
            g0.wait()
            pltpu.sync_copy(rows0, agg_sh.at[sd0.at[1]], add=True)
            if with_deg:
                deg_update(sd0)

            @pl.when(i + 2 < NCHUNK)
            def _():
                pltpu.async_copy(sd_hbm.at[wid].at[i + 2], sd0, isem0)

            g1.wait()
            pltpu.sync_copy(rows1, agg_sh.at[sd1.at[1]], add=True)
            if with_deg:
                deg_update(sd1)

            @pl.when(i + 2 < NCHUNK)
            def _():
                pltpu.make_async_copy(sd_hbm.at[wid].at[i + 2], sd0,
                                      isem0).wait()
                pltpu.async_copy(x_hbm.at[sd0.at[0]], rows0, gsem0)

            @pl.when(i + 3 < NCHUNK)
            def _():
                pltpu.async_copy(sd_hbm.at[wid].at[i + 3], sd1, isem1)
                pltpu.make_async_copy(sd_hbm.at[wid].at[i + 3], sd1,
                                      isem1).wait()
                pltpu.async_copy(x_hbm.at[sd1.at[0]], rows1, gsem1)

        plsc.subcore_barrier()

        def wdrain():
            pltpu.make_async_copy(
                zbuf, agg_out.at[pl.ds(cid * N, SUB)], wsem).wait()

        @pl.loop(0, NSUB, step=2)
        def _(j):
            @pl.when(j > 0)
            def _():
                wdrain()
            r0 = sid * RPT + j * SUB
            pltpu.sync_copy(agg_sh.at[pl.ds(r0, SUB)], zbuf)
            pltpu.async_copy(zbuf, agg_out.at[pl.ds(cid * N + r0, SUB)],
                             wsem)
            r1 = r0 + SUB
            pltpu.sync_copy(agg_sh.at[pl.ds(r1, SUB)], zbuf2)
            wdrain()
            pltpu.async_copy(zbuf2, agg_out.at[pl.ds(cid * N + r1, SUB)],
                             wsem)

        wdrain()

        @pl.when(sid == 0)
        def _():
            t0 = NS * RPT
            pltpu.sync_copy(agg_sh.at[pl.ds(t0, TAIL)],
                            zbuf.at[pl.ds(0, TAIL)])
            pltpu.sync_copy(zbuf.at[pl.ds(0, TAIL)],
                            agg_out.at[pl.ds(cid * N + t0, TAIL)])

        if with_deg:
            pltpu.sync_copy(degloc.at[pl.ds(0, N)], deg_out.at[wid])

    if with_deg:
        def body_w(x_hbm, sd_hbm, zrow_hbm, agg_out, deg_out,
                   agg_sh, sd0, sd1, rows0, rows1, zbuf, zbuf2,
                   gsem0, gsem1, isem0, isem1, wsem, degloc):
            body(x_hbm, sd_hbm, zrow_hbm, agg_out, deg_out,
                 agg_sh, sd0, sd1, rows0, rows1, zbuf, zbuf2,
                 gsem0, gsem1, isem0, isem1, wsem, degloc)
    else:
        def body_w(x_hbm, sd_hbm, zrow_hbm, agg_out,
                   agg_sh, sd0, sd1, rows0, rows1, zbuf, zbuf2,
                   gsem0, gsem1, isem0, isem1, wsem):
            body(x_hbm, sd_hbm, zrow_hbm, agg_out, None,
                 agg_sh, sd0, sd1, rows0, rows1, zbuf, zbuf2,
                 gsem0, gsem1, isem0, isem1, wsem)

    return pl.kernel(body_w, out_type=tuple(out_type), mesh=mesh,
                     scratch_types=scratch,
                     compiler_params=pltpu.CompilerParams(
                         use_tc_tiling_on_sc=False,
                         needs_layout_passes=False))


_sc_agg_deg = _make_sc_agg(True)
_sc_agg = _make_sc_agg(False)


_TC_R = 2000


def _tc_body(agg_ref, deg_ref, x_ref, w_ref, b_ref, o_ref):
    a = agg_ref[0] + agg_ref[1]
    dg = jnp.sum(deg_ref[...], axis=1, keepdims=True)
    a = a / jnp.maximum(dg, 1.0)
    h = jnp.dot(a, w_ref[...], preferred_element_type=jnp.float32)
    o_ref[...] = jnp.maximum(h + b_ref[...], 0.0) + x_ref[...]


def _tc_layer(agg, degT, x, W, b):
    return pl.pallas_call(
        _tc_body,
        out_shape=jax.ShapeDtypeStruct((N, D), jnp.float32),
        grid=(N // _TC_R,),
        in_specs=[
            pl.BlockSpec((NC, _TC_R, D), lambda i: (0, i, 0)),
            pl.BlockSpec((_TC_R, NW), lambda i: (i, 0)),
            pl.BlockSpec((_TC_R, D), lambda i: (i, 0)),
            pl.BlockSpec((D, D), lambda i: (0, 0)),
            pl.BlockSpec((1, D), lambda i: (0, 0)),
        ],
        out_specs=pl.BlockSpec((_TC_R, D), lambda i: (i, 0)),
    )(agg, degT, x, W, b)


def kernel(x, edge_index, W1, b1, W2, b2):
    ei = edge_index.astype(jnp.int32).reshape(2, NW, NCHUNK, C)
    sd = ei.transpose(1, 2, 0, 3)
    pad = jnp.concatenate(
        [jnp.zeros((NW, NCHUNK, 1, CP - C), jnp.int32),
         jnp.full((NW, NCHUNK, 1, CP - C), N, jnp.int32)], axis=2)
    sdp = jnp.concatenate([sd, pad], axis=3)
    zrow = jnp.zeros((SUB, D), jnp.float32)
    b1r = b1.reshape(1, D)
    b2r = b2.reshape(1, D)

    agg1, degp = _sc_agg_deg(x, sdp, zrow)
    agg1 = agg1.reshape(NC, N, D)
    degT = degp.T
    h1 = _tc_layer(agg1, degT, x, W1, b1r)

    (agg2,) = _sc_agg(h1, sdp, zrow)
    agg2 = agg2.reshape(NC, N, D)
    out = _tc_layer(agg2, degT, h1, W2, b2r)
    return out


# speedup vs baseline: 2.3712x; 2.3712x over previous
import jax
import jax.numpy as jnp
from jax import lax
from jax.experimental import pallas as pl
from jax.experimental.pallas import tpu as pltpu
from jax.experimental.pallas import tpu_sc as plsc

N = 10000
D = 128
E = 320000
NC = 2
NS = 16
NW = NC * NS
EPW = E // NW
C = 125
CP = 128
NCHUNK = EPW // C
NP = N + NS
ND = 10016
RPT = 624
TAIL = N - NS * RPT
SUB = 24
NSUB = RPT // SUB
CV = CP // 16


def _make_sc_agg(with_deg):
    out_type = [jax.ShapeDtypeStruct((NC * N, D), jnp.float32)]
    scratch = [
        pltpu.VMEM_SHARED((NP, D), jnp.float32),
        pltpu.VMEM((2, CP), jnp.int32),
        pltpu.VMEM((2, CP), jnp.int32),
        pltpu.VMEM((CP, D), jnp.float32),
        pltpu.VMEM((CP, D), jnp.float32),
        pltpu.VMEM((SUB, D), jnp.float32),
        pltpu.VMEM((SUB, D), jnp.float32),
        pltpu.SemaphoreType.DMA,
        pltpu.SemaphoreType.DMA,
        pltpu.SemaphoreType.DMA,
        pltpu.SemaphoreType.DMA,
        pltpu.SemaphoreType.DMA,
    ]
    if with_deg:
        out_type.append(jax.ShapeDtypeStruct((NW, N), jnp.float32))
        scratch.append(pltpu.VMEM((ND,), jnp.float32))

    mesh = plsc.VectorSubcoreMesh(core_axis_name="c", subcore_axis_name="s")

    def body(x_hbm, sd_hbm, zrow_hbm,
             agg_out, deg_out, agg_sh, sd0, sd1,
             rows0, rows1, zbuf, zbuf2, gsem0, gsem1, isem0, isem1,
             wsem, degloc=None):
        cid = lax.axis_index("c")
        sid = lax.axis_index("s")
        wid = sid * NC + cid

        pltpu.sync_copy(sd_hbm.at[wid].at[0], sd0)
        g0 = pltpu.async_copy(x_hbm.at[sd0.at[0]], rows0, gsem0)
        pltpu.sync_copy(sd_hbm.at[wid].at[1], sd1)
        g1 = pltpu.async_copy(x_hbm.at[sd1.at[0]], rows1, gsem1)

        pltpu.sync_copy(zrow_hbm, zbuf)

        @pl.loop(0, NSUB)
        def _(j):
            pltpu.async_copy(zbuf, agg_sh.at[pl.ds(sid * RPT + j * SUB, SUB)],
                             wsem)

        @pl.when(sid == 0)
        def _():
            pltpu.sync_copy(zbuf.at[pl.ds(0, TAIL)],
                            agg_sh.at[pl.ds(NS * RPT, TAIL)])

        if with_deg:
            zv = jnp.zeros((16,), jnp.float32)

            @pl.loop(0, ND // 16)
            def _(k):
                degloc[pl.ds(k * 16, 16)] = zv

        @pl.loop(0, NSUB)
        def _(j):
            pltpu.make_async_copy(
                zbuf, agg_sh.at[pl.ds(sid * RPT, SUB)], wsem).wait()

        plsc.subcore_barrier()

        ones16 = jnp.ones((16,), jnp.float32)

        def deg_update(sd):
            for g in range(CV):
                idx = sd[1, pl.ds(g * 16, 16)]
                plsc.addupdate_scatter(degloc, [idx], ones16)

        @pl.loop(0, NCHUNK, step=2)
        def _(i):
            g0.wait()
            pltpu.sync_copy(rows0, agg_sh.at[sd0.at[1]], add=True)
            if with_deg:
                deg_update(sd0)

            @pl.when(i + 2 < NCHUNK)
            def _():
                pltpu.async_copy(sd_hbm.at[wid].at[i + 2], sd0, isem0)

            g1.wait()
            pltpu.sync_copy(rows1, agg_sh.at[sd1.at[1]], add=True)
            if with_deg:
                deg_update(sd1)

            @pl.when(i + 2 < NCHUNK)
            def _():
                pltpu.make_async_copy(sd_hbm.at[wid].at[i + 2], sd0,
                                      isem0).wait()
                pltpu.async_copy(x_hbm.at[sd0.at[0]], rows0, gsem0)

            @pl.when(i + 3 < NCHUNK)
            def _():
                pltpu.async_copy(sd_hbm.at[wid].at[i + 3], sd1, isem1)
                pltpu.make_async_copy(sd_hbm.at[wid].at[i + 3], sd1,
                                      isem1).wait()
                pltpu.async_copy(x_hbm.at[sd1.at[0]], rows1, gsem1)

        plsc.subcore_barrier()

        def wdrain():
            pltpu.make_async_copy(
                zbuf, agg_out.at[pl.ds(cid * N, SUB)], wsem).wait()

        @pl.loop(0, NSUB, step=2)
        def _(j):
            @pl.when(j > 0)
            def _():
                wdrain()
            r0 = sid * RPT + j * SUB
            pltpu.sync_copy(agg_sh.at[pl.ds(r0, SUB)], zbuf)
            pltpu.async_copy(zbuf, agg_out.at[pl.ds(cid * N + r0, SUB)],
                             wsem)
            r1 = r0 + SUB
            pltpu.sync_copy(agg_sh.at[pl.ds(r1, SUB)], zbuf2)
            wdrain()
            pltpu.async_copy(zbuf2, agg_out.at[pl.ds(cid * N + r1, SUB)],
                             wsem)

        wdrain()

        @pl.when(sid == 0)
        def _():
            t0 = NS * RPT
            pltpu.sync_copy(agg_sh.at[pl.ds(t0, TAIL)],
                            zbuf.at[pl.ds(0, TAIL)])
            pltpu.sync_copy(zbuf.at[pl.ds(0, TAIL)],
                            agg_out.at[pl.ds(cid * N + t0, TAIL)])

        if with_deg:
            pltpu.sync_copy(degloc.at[pl.ds(0, N)], deg_out.at[wid])

    if with_deg:
        def body_w(x_hbm, sd_hbm, zrow_hbm, agg_out, deg_out,
                   agg_sh, sd0, sd1, rows0, rows1, zbuf, zbuf2,
                   gsem0, gsem1, isem0, isem1, wsem, degloc):
            body(x_hbm, sd_hbm, zrow_hbm, agg_out, deg_out,
                 agg_sh, sd0, sd1, rows0, rows1, zbuf, zbuf2,
                 gsem0, gsem1, isem0, isem1, wsem, degloc)
    else:
        def body_w(x_hbm, sd_hbm, zrow_hbm, agg_out,
                   agg_sh, sd0, sd1, rows0, rows1, zbuf, zbuf2,
                   gsem0, gsem1, isem0, isem1, wsem):
            body(x_hbm, sd_hbm, zrow_hbm, agg_out, None,
                 agg_sh, sd0, sd1, rows0, rows1, zbuf, zbuf2,
                 gsem0, gsem1, isem0, isem1, wsem)

    return pl.kernel(body_w, out_type=tuple(out_type), mesh=mesh,
                     scratch_types=scratch,
                     compiler_params=pltpu.CompilerParams(
                         use_tc_tiling_on_sc=False,
                         needs_layout_passes=False))


_sc_agg_deg = _make_sc_agg(True)
_sc_agg = _make_sc_agg(False)


_TC_R = 2000


def _tc_body(agg_ref, deg_ref, x_ref, w_ref, b_ref, o_ref):
    a = agg_ref[0] + agg_ref[1]
    dg = jnp.sum(deg_ref[...], axis=1, keepdims=True)
    a = a / jnp.maximum(dg, 1.0)
    h = jnp.dot(a, w_ref[...], preferred_element_type=jnp.float32)
    o_ref[...] = jnp.maximum(h + b_ref[...], 0.0) + x_ref[...]


def _tc_layer(agg, degT, x, W, b):
    return pl.pallas_call(
        _tc_body,
        out_shape=jax.ShapeDtypeStruct((N, D), jnp.float32),
        grid=(N // _TC_R,),
        in_specs=[
            pl.BlockSpec((NC, _TC_R, D), lambda i: (0, i, 0)),
            pl.BlockSpec((_TC_R, NW), lambda i: (i, 0)),
            pl.BlockSpec((_TC_R, D), lambda i: (i, 0)),
            pl.BlockSpec((D, D), lambda i: (0, 0)),
            pl.BlockSpec((1, D), lambda i: (0, 0)),
        ],
        out_specs=pl.BlockSpec((_TC_R, D), lambda i: (i, 0)),
    )(agg, degT, x, W, b)


def kernel(x, edge_index, W1, b1, W2, b2):
    ei = edge_index.astype(jnp.int32).reshape(2, NW, NCHUNK, C)
    sd = ei.transpose(1, 2, 0, 3)
    tile = (jnp.arange(NW, dtype=jnp.int32) // NC)[:, None, None, None]
    pad_src = jnp.broadcast_to(tile * 600, (NW, NCHUNK, 1, CP - C))
    pad_dst = jnp.broadcast_to(N + tile, (NW, NCHUNK, 1, CP - C))
    pad = jnp.concatenate([pad_src, pad_dst], axis=2)
    sdp = jnp.concatenate([sd, pad], axis=3)
    zrow = jnp.zeros((SUB, D), jnp.float32)
    b1r = b1.reshape(1, D)
    b2r = b2.reshape(1, D)

    agg1, degp = _sc_agg_deg(x, sdp, zrow)
    agg1 = agg1.reshape(NC, N, D)
    degT = degp.T
    h1 = _tc_layer(agg1, degT, x, W1, b1r)

    (agg2,) = _sc_agg(h1, sdp, zrow)
    agg2 = agg2.reshape(NC, N, D)
    out = _tc_layer(agg2, degT, h1, W2, b2r)
    return out
